# Initial kernel scaffold; baseline (speedup 1.0000x reference)
import jax
import jax.numpy as jnp
from jax.experimental import pallas as pl


def kernel(x, W1f, W1d, W2f, W2d, W3f, W3d, W4f, W4d, W5f, W5d):
    raise NotImplementedError("write your pallas kernel here")


# speedup vs baseline: 1.0104x; 1.0104x over previous
import functools

import jax
import jax.numpy as jnp
import numpy as np
from jax.experimental import pallas as pl
from jax.experimental.pallas import tpu as pltpu

EPS = 1e-6
NS = 0.2
B, N, K = 8, 1024, 20


def _final_block_body(cat_ref, wf_ref, wd_ref, out_ref):
    c0 = cat_ref[0]
    c1 = cat_ref[1]
    c2 = cat_ref[2]
    wf = wf_ref[:]
    wd = wd_ref[:]
    p0 = jax.lax.dot(wf, c0, preferred_element_type=jnp.float32)
    p1 = jax.lax.dot(wf, c1, preferred_element_type=jnp.float32)
    p2 = jax.lax.dot(wf, c2, preferred_element_type=jnp.float32)
    d0 = jax.lax.dot(wd, c0, preferred_element_type=jnp.float32)
    d1 = jax.lax.dot(wd, c1, preferred_element_type=jnp.float32)
    d2 = jax.lax.dot(wd, c2, preferred_element_type=jnp.float32)
    norm = jnp.sqrt(p0 * p0 + p1 * p1 + p2 * p2) + EPS
    m = jnp.mean(norm, axis=1, keepdims=True)
    v = jnp.mean(norm * norm, axis=1, keepdims=True) - m * m
    nbn = (norm - m) / jnp.sqrt(v + 1e-5)
    scale = nbn / norm
    q0 = p0 * scale
    q1 = p1 * scale
    q2 = p2 * scale
    dot = q0 * d0 + q1 * d1 + q2 * d2
    dsq = d0 * d0 + d1 * d1 + d2 * d2
    coef = (1.0 - NS) * jnp.where(dot < 0, dot / (dsq + EPS), 0.0)
    out_ref[0] = q0 - coef * d0
    out_ref[1] = q1 - coef * d1
    out_ref[2] = q2 - coef * d2


def _final_block(cat3, wf, wd, block_o=64):
    _, C, M = cat3.shape
    O = wf.shape[0]
    nob = pl.cdiv(O, block_o)
    out = pl.pallas_call(
        _final_block_body,
        grid=(nob,),
        in_specs=[
            pl.BlockSpec((3, C, M), lambda i: (0, 0, 0)),
            pl.BlockSpec((block_o, C), lambda i: (i, 0)),
            pl.BlockSpec((1, C), lambda i: (0, 0)),
        ],
        out_specs=pl.BlockSpec((3, block_o, M), lambda i: (0, i, 0)),
        out_shape=jax.ShapeDtypeStruct((3, O, M), jnp.float32),
    )(cat3, wf, wd)
    return out


def _knn_idx(yf):
    inner = -2.0 * jnp.einsum('bdn,bdm->bnm', yf, yf)
    xx = jnp.sum(yf * yf, axis=1)
    pd = -xx[:, :, None] - inner - xx[:, None, :]
    return jax.lax.top_k(pd, K)[1]


def _edge_block(y, idx, wf, wd):
    Bb, C, _, Nn = y.shape
    yf = y.reshape(Bb, C * 3, Nn)
    yt = jnp.transpose(yf, (0, 2, 1))
    feat = jax.vmap(lambda pts, ind: pts[ind])(yt, idx)
    feat = feat.reshape(Bb, Nn, K, C, 3)
    xe = yt.reshape(Bb, Nn, 1, C, 3)
    f = jnp.concatenate([feat - xe, jnp.broadcast_to(xe, feat.shape)], axis=3)
    f = jnp.transpose(f, (0, 3, 4, 1, 2))
    p = jnp.einsum('oc,bcvnk->bovnk', wf, f)
    d = jnp.einsum('oc,bcvnk->bovnk', wd, f)
    norm = jnp.sqrt(jnp.sum(p * p, axis=2)) + EPS
    axes = (0, 2, 3)
    m = jnp.mean(norm, axis=axes, keepdims=True)
    v = jnp.var(norm, axis=axes, keepdims=True)
    nbn = (norm - m) / jnp.sqrt(v + 1e-5)
    ph = p / norm[:, :, None] * nbn[:, :, None]
    dot = jnp.sum(ph * d, axis=2, keepdims=True)
    dsq = jnp.sum(d * d, axis=2, keepdims=True)
    out = ph - (1.0 - NS) * jnp.where(dot < 0, dot / (dsq + EPS), 0.0) * d
    return out.mean(axis=-1)


def kernel(x, W1f, W1d, W2f, W2d, W3f, W3d, W4f, W4d, W5f, W5d):
    h = jnp.transpose(x, (0, 2, 1))[:, None, :, :]
    y = h
    x1 = _edge_block(y, _knn_idx(y.reshape(B, -1, N)), W1f, W1d)
    x2 = _edge_block(x1, _knn_idx(x1.reshape(B, -1, N)), W2f, W2d)
    x3 = _edge_block(x2, _knn_idx(x2.reshape(B, -1, N)), W3f, W3d)
    x4 = _edge_block(x3, _knn_idx(x3.reshape(B, -1, N)), W4f, W4d)
    cat = jnp.concatenate([x1, x2, x3, x4], axis=1)
    cat3 = jnp.transpose(cat, (2, 1, 0, 3)).reshape(3, 169, B * N)
    out3 = _final_block(cat3, W5f, W5d)
    out = jnp.transpose(out3.reshape(3, 341, B, N), (2, 1, 0, 3))
    return out


# speedup vs baseline: 3.1046x; 3.0728x over previous
import functools

import jax
import jax.numpy as jnp
import numpy as np
from jax.experimental import pallas as pl
from jax.experimental.pallas import tpu as pltpu

EPS = 1e-6
NS = 0.2
B, N, K = 8, 1024, 20
NEG = np.float32(-3.0e38)


def _pad_rows(w, cp):
    o, c = w.shape
    return jnp.pad(w, ((0, 0), (0, cp - c)))


def _knn_body(yt_ref, yf_ref, xxi_ref, xxj_ref, out_ref, *, nb, k):
    b = pl.program_id(0)
    yt = yt_ref[0]
    yf = yf_ref[0]
    g = jax.lax.dot(yt, yf, preferred_element_type=jnp.float32)
    xxj = xxj_ref[0]
    xxi = xxi_ref[0]
    pd = 2.0 * g - xxi - xxj
    cols = jax.lax.broadcasted_iota(jnp.int32, pd.shape, 1)
    kcols = jax.lax.broadcasted_iota(jnp.int32, (nb, k), 1)

    def body(j, carry):
        pd, idxs = carry
        m = jnp.max(pd, axis=1, keepdims=True)
        sel = pd >= m
        idxj = jnp.min(jnp.where(sel, cols, N), axis=1, keepdims=True)
        idxs = jnp.where(kcols == j, idxj, idxs)
        pd = jnp.where(cols == idxj, NEG, pd)
        return pd, idxs

    idxs0 = jnp.zeros((nb, k), jnp.int32)
    _, idxs = jax.lax.fori_loop(0, k, body, (pd, idxs0))
    out_ref[0] = idxs + b * N


def _knn(table, tablet, xx, nb=256):
    _, _, W = table.shape
    grid = (B, N // nb)
    return pl.pallas_call(
        functools.partial(_knn_body, nb=nb, k=K),
        grid=grid,
        in_specs=[
            pl.BlockSpec((1, nb, W), lambda b, i: (b, i, 0)),
            pl.BlockSpec((1, W, N), lambda b, i: (b, 0, 0)),
            pl.BlockSpec((1, nb, 1), lambda b, i: (b, i, 0)),
            pl.BlockSpec((1, 1, N), lambda b, i: (b, 0, 0)),
        ],
        out_specs=pl.BlockSpec((1, nb, K), lambda b, i: (b, i, 0)),
        out_shape=jax.ShapeDtypeStruct((B, N, K), jnp.int32),
    )(table, tablet, xx.reshape(B, N, 1), xx.reshape(B, 1, N))


def _edge_vecs(gt, ct, wcat, cp, o, e, k):
    out = []
    nb = e // k
    for v in range(3):
        xv = gt[v * cp:(v + 1) * cp, :]
        cv = ct[v * cp:(v + 1) * cp, :]
        cv_rep = jnp.broadcast_to(cv[:, :, None], (cp, nb, k)).reshape(cp, e)
        feat = jnp.concatenate([xv - cv_rep, cv_rep], axis=0)
        out.append(jax.lax.dot(wcat, feat,
                               preferred_element_type=jnp.float32))
    return out


def _passA_body(g_ref, c_ref, wf_ref, norms_ref, *, cp, o, e, k):
    gt = jnp.transpose(g_ref[0])
    ct = jnp.transpose(c_ref[0])
    p0, p1, p2 = _edge_vecs(gt, ct, wf_ref[:], cp, o, e, k)
    norm = jnp.sqrt(p0 * p0 + p1 * p1 + p2 * p2) + EPS
    norms_ref[0] = norm


def _passB_body(g_ref, c_ref, wf_ref, wd_ref, stats_ref,
                var_ref, out_ref, *, cp, o, e, k):
    gt = jnp.transpose(g_ref[0])
    ct = jnp.transpose(c_ref[0])
    p0, p1, p2 = _edge_vecs(gt, ct, wf_ref[:], cp, o, e, k)
    d0, d1, d2 = _edge_vecs(gt, ct, wd_ref[:], cp, o, e, k)
    m = stats_ref[...]
    var = var_ref[...]
    norm = jnp.sqrt(p0 * p0 + p1 * p1 + p2 * p2) + EPS
    nbn = (norm - m) / jnp.sqrt(var + 1e-5)
    q0 = (p0 / norm) * nbn
    q1 = (p1 / norm) * nbn
    q2 = (p2 / norm) * nbn
    dot = q0 * d0 + q1 * d1 + q2 * d2
    dsq = d0 * d0 + d1 * d1 + d2 * d2
    mask = (dot >= 0).astype(jnp.float32)
    c = dot / (dsq + EPS)
    nb = e // k
    for v, (q, d) in enumerate(((q0, d0), (q1, d1), (q2, d2))):
        r = NS * q + (1.0 - NS) * (mask * q + (1.0 - mask) * (q - c * d))
        pooled = jnp.mean(r.reshape(o, nb, k), axis=2)
        out_ref[0, v] = pooled


def _edge_block(gathered, centers, wf, wd, C, O, nb=128):
    cp = C
    W = centers.shape[-1]
    e = nb * K
    wfcat = wf
    wdcat = wd
    grid = (B, N // nb)
    g_spec = pl.BlockSpec((1, e, W), lambda b, i: (b, i, 0))
    c_spec = pl.BlockSpec((1, nb, W), lambda b, i: (b, i, 0))
    w_spec = pl.BlockSpec((O, 2 * C), lambda b, i: (0, 0))
    s_spec = pl.BlockSpec((O, 1), lambda b, i: (0, 0))
    n_spec = pl.BlockSpec((1, O, e), lambda b, i: (b, 0, i))
    norms = pl.pallas_call(
        functools.partial(_passA_body, cp=cp, o=O, e=e, k=K),
        grid=grid,
        in_specs=[g_spec, c_spec, w_spec],
        out_specs=n_spec,
        out_shape=jax.ShapeDtypeStruct((B, O, N * K), jnp.float32),
    )(gathered, centers, wfcat)
    n4 = norms.reshape(B, O, N, K)
    stats = jnp.mean(n4, axis=(0, 2, 3)).reshape(O, 1)
    var = jnp.var(n4, axis=(0, 2, 3)).reshape(O, 1)
    out = pl.pallas_call(
        functools.partial(_passB_body, cp=cp, o=O, e=e, k=K),
        grid=grid,
        in_specs=[g_spec, c_spec, w_spec, w_spec, s_spec, s_spec],
        out_specs=pl.BlockSpec((1, 3, O, nb), lambda b, i: (b, 0, 0, i)),
        out_shape=jax.ShapeDtypeStruct((B, 3, O, N), jnp.float32),
    )(gathered, centers, wfcat, wdcat, stats, var)
    return out


def _table_from_x3(x3, cpad, wrow):
    b, _, c, n = x3.shape
    xp = jnp.pad(x3, ((0, 0), (0, 0), (0, cpad - c), (0, 0)))
    rows = jnp.transpose(xp, (0, 3, 1, 2)).reshape(b, n, 3 * cpad)
    if wrow > 3 * cpad:
        rows = jnp.pad(rows, ((0, 0), (0, 0), (0, wrow - 3 * cpad)))
    return rows


def _gather_rows(table, idx):
    W = table.shape[-1]
    flat = table.reshape(B * N, W)
    g = jnp.take(flat, idx.reshape(-1), axis=0)
    return g.reshape(B, N * K, W)


def _sc_gather(table_flat, idx, ch=128):
    from jax.experimental.pallas import tpu_sc as plsc
    from jax import lax

    M = idx.size
    W = table_flat.shape[-1]
    info = plsc.get_sparse_core_info()
    nw = info.num_cores * info.num_subcores
    per_w = M // nw
    n_ch = per_w // ch
    assert per_w % ch == 0
    idx3 = idx.reshape(nw, n_ch, ch)
    nc = info.num_cores
    mesh = plsc.VectorSubcoreMesh(core_axis_name="c", subcore_axis_name="s")

    @functools.partial(
        pl.kernel, mesh=mesh,
        out_type=jax.ShapeDtypeStruct((M, W), jnp.float32),
        scratch_types=[
            pltpu.VMEM((ch,), jnp.int32),
            pltpu.VMEM((ch, W), jnp.float32),
            pltpu.SemaphoreType.DMA,
        ],
    )
    def k(table_hbm, idx_hbm, out_hbm, idx_v, rows_v, sem):
        wid = lax.axis_index("s") * nc + lax.axis_index("c")

        def step(i, carry):
            pltpu.sync_copy(idx_hbm.at[wid, i], idx_v)
            pltpu.async_copy(table_hbm.at[idx_v], rows_v, sem).wait()
            base = (wid * n_ch + i) * ch
            pltpu.sync_copy(rows_v, out_hbm.at[pl.ds(base, ch)])
            return carry

        lax.fori_loop(0, n_ch, step, 0)

    return k(table_flat, idx3)


def _stage(x3, C, O, wf, wd, wrow, nb=128):
    table = _table_from_x3(x3, C, wrow)
    xf = jnp.transpose(x3, (0, 2, 1, 3)).reshape(B, C * 3, N)
    xx = jnp.sum(xf * xf, axis=1)
    dpad = ((C * 3 + 7) // 8) * 8
    xfp = jnp.pad(xf, ((0, 0), (0, dpad - C * 3), (0, 0)))
    ktable = jnp.transpose(xfp, (0, 2, 1))
    idx = _knn(ktable, xfp, xx)
    W = table.shape[-1]
    gathered = _sc_gather(table.reshape(B * N, W), idx.reshape(-1))
    gathered = gathered.reshape(B, N * K, W)
    return _edge_block(gathered, table, wf, wd, C, O, nb=nb)


def _final_mm_body(cat_ref, wf_ref, wd_ref, p_ref, d_ref, st_ref):
    first = pl.program_id(0) == 0
    wf = wf_ref[:]
    wd = wd_ref[:]
    norm2 = None
    for v in range(3):
        cv = cat_ref[v]
        pv = jax.lax.dot(wf, cv, preferred_element_type=jnp.float32)
        dv = jax.lax.dot(wd, cv, preferred_element_type=jnp.float32)
        p_ref[v] = pv
        d_ref[v] = dv
        norm2 = pv * pv if norm2 is None else norm2 + pv * pv
    norm = jnp.sqrt(norm2) + EPS
    s1 = jnp.sum(norm, axis=1, keepdims=True)
    s2 = jnp.sum(norm * norm, axis=1, keepdims=True)
    st = jnp.concatenate([s1, s2], axis=1)

    @pl.when(first)
    def _():
        st_ref[...] = jnp.zeros_like(st_ref)

    st_ref[...] += st


def _final_apply_body(p_ref, d_ref, st_ref, out_ref, *, m_total):
    cnt = jnp.float32(m_total)
    m = st_ref[:, 0:1] / cnt
    var = st_ref[:, 1:2] / cnt - m * m
    p0, p1, p2 = p_ref[0], p_ref[1], p_ref[2]
    d0, d1, d2 = d_ref[0], d_ref[1], d_ref[2]
    norm = jnp.sqrt(p0 * p0 + p1 * p1 + p2 * p2) + EPS
    scale = (norm - m) / (jnp.sqrt(var + 1e-5) * norm)
    q0 = p0 * scale
    q1 = p1 * scale
    q2 = p2 * scale
    dot = q0 * d0 + q1 * d1 + q2 * d2
    dsq = d0 * d0 + d1 * d1 + d2 * d2
    coef = (1.0 - NS) * jnp.where(dot < 0, dot / (dsq + EPS), 0.0)
    out_ref[0] = q0 - coef * d0
    out_ref[1] = q1 - coef * d1
    out_ref[2] = q2 - coef * d2


def _final_block(cat3, wf, wd, mb=1024):
    _, C, M = cat3.shape
    O = wf.shape[0]
    grid = (M // mb,)
    p, d, st = pl.pallas_call(
        _final_mm_body,
        grid=grid,
        in_specs=[
            pl.BlockSpec((3, C, mb), lambda i: (0, 0, i)),
            pl.BlockSpec((O, C), lambda i: (0, 0)),
            pl.BlockSpec((1, C), lambda i: (0, 0)),
        ],
        out_specs=[
            pl.BlockSpec((3, O, mb), lambda i: (0, 0, i)),
            pl.BlockSpec((3, 1, mb), lambda i: (0, 0, i)),
            pl.BlockSpec((O, 2), lambda i: (0, 0)),
        ],
        out_shape=[
            jax.ShapeDtypeStruct((3, O, M), jnp.float32),
            jax.ShapeDtypeStruct((3, 1, M), jnp.float32),
            jax.ShapeDtypeStruct((O, 2), jnp.float32),
        ],
    )(cat3, wf, wd)
    return pl.pallas_call(
        functools.partial(_final_apply_body, m_total=M),
        grid=grid,
        in_specs=[
            pl.BlockSpec((3, O, mb), lambda i: (0, 0, i)),
            pl.BlockSpec((3, 1, mb), lambda i: (0, 0, i)),
            pl.BlockSpec((O, 2), lambda i: (0, 0)),
        ],
        out_specs=pl.BlockSpec((3, O, mb), lambda i: (0, 0, i)),
        out_shape=jax.ShapeDtypeStruct((3, O, M), jnp.float32),
    )(p, d, st)


def kernel(x, W1f, W1d, W2f, W2d, W3f, W3d, W4f, W4d, W5f, W5d):
    h = jnp.transpose(x, (0, 2, 1))[:, :, None, :]
    x1 = _stage(h, 1, 21, W1f, W1d, wrow=128)
    x2 = _stage(x1, 21, 21, W2f, W2d, wrow=128)
    x3 = _stage(x2, 21, 42, W3f, W3d, wrow=128)
    x4 = _stage(x3, 42, 85, W4f, W4d, wrow=128)
    cat = jnp.concatenate([x1, x2, x3, x4], axis=2)
    cat3 = jnp.transpose(cat, (1, 2, 0, 3)).reshape(3, 169, B * N)
    out3 = _final_block(cat3, W5f, W5d)
    out = jnp.transpose(out3.reshape(3, 341, B, N), (2, 1, 0, 3))
    return out
